# baseline (speedup 1.0000x reference)
import math

import jax
import jax.numpy as jnp
from jax import lax
from jax.experimental import pallas as pl
from jax.experimental.pallas import tpu as pltpu
from jax.experimental.pallas import tpu_sc as plsc

_DIM = 64
_SCALE = math.sqrt(_DIM)
_NC = 2
_NS = 16
_NW = _NC * _NS
_LANES = 16
_B, _S = 4096, 200
_ST, _BT = _S // 8, _B // 128
_RQ = 256
_NU = _ST * _BT * 4
_UPW = _NU // _NW
_TILE = 1024


def _make_kernel():
    mesh = plsc.VectorSubcoreMesh(core_axis_name="c", subcore_axis_name="s")

    def body(xf, table2, outf,
             idx0, idx1, pidx0, pidx1, par0, par1, rows0, rows1,
             stg0, stg1, stg2, stg3,
             isem0, isem1, gsem0, gsem1, ssem0, ssem1, ssem2, ssem3):
        wid = lax.axis_index("s") * _NC + lax.axis_index("c")
        idxb = (idx0, idx1)
        pidxb = (pidx0, pidx1)
        parb = (par0, par1)
        rowsb = (rows0, rows1)
        isem = (isem0, isem1)
        gsem = (gsem0, gsem1)
        stg = (stg0, stg1, stg2, stg3)
        ssem = (ssem0, ssem1, ssem2, ssem3)
        iota = lax.iota(jnp.int32, _LANES)

        def idx_copy(u, p):
            h = wid * _UPW + u
            return pltpu.make_async_copy(
                xf.at[pl.ds(h * _RQ, _RQ)], idxb[p], isem[p]
            )

        def prep(p):
            @plsc.parallel_loop(0, _RQ // _LANES, unroll=4)
            def _c(c):
                off = pl.ds(c * _LANES, _LANES)
                v = idxb[p][off]
                pidxb[p][off] = lax.shift_right_logical(v, 1)
                parb[p][off] = lax.shift_left(lax.bitwise_and(v, 1), 6)

        def gather(p):
            return pltpu.make_async_copy(
                table2.at[pidxb[p]], rowsb[p], gsem[p]
            )

        def store_dmas(u, sl, buf):
            h = wid * _UPW + u
            st = h // (_BT * 4)
            rem = h % (_BT * 4)
            bt = rem // 4
            q = rem % 4
            s_glob = st * 8 + q * 2 + sl
            woff = s_glob * (8 * _BT * _TILE) + bt * _TILE
            return [
                pltpu.make_async_copy(
                    stg[buf].at[pl.ds(dt * _TILE, _TILE)],
                    outf.at[pl.ds(woff + dt * (_BT * _TILE), _TILE)],
                    ssem[buf],
                )
                for dt in range(8)
            ]

        idx_copy(0, 0).start()
        idx_copy(1, 1).start()
        idx_copy(0, 0).wait()
        prep(0)
        gather(0).start()

        @pl.loop(0, _UPW, step=2)
        def _ring(g):
            for p in range(2):
                u = g + p
                gather(p).wait()

                @pl.when(u + 2 < _UPW)
                def _():
                    idx_copy(u + 2, p).start()

                @pl.when(u + 1 < _UPW)
                def _():
                    idx_copy(u + 1, 1 - p).wait()
                    prep(1 - p)
                    gather(1 - p).start()

                rows = rowsb[p]
                for sl in range(2):
                    buf = p * 2 + sl

                    @pl.when(u >= 2)
                    def _():
                        for d in store_dmas(u - 2, sl, buf):
                            d.wait()

                    @plsc.parallel_loop(0, _RQ // _LANES // 2, unroll=1)
                    def _grp(grp):
                        off = sl * 128 + grp * _LANES
                        rid = iota + off
                        pvec = parb[p][pl.ds(off, _LANES)]
                        for dt in range(8):
                            for jl in range(8):
                                cid = pvec + (dt * 8 + jl)
                                v = plsc.load_gather(rows, [rid, cid])
                                stg[buf][
                                    pl.ds(dt * _TILE + jl * 128 + grp * _LANES,
                                          _LANES)
                                ] = v * _SCALE

                    for d in store_dmas(u, sl, buf):
                        d.start()

        for p in range(2):
            for sl in range(2):
                for d in store_dmas(_UPW - 2 + p, sl, p * 2 + sl):
                    d.wait()

    return pl.kernel(
        body,
        out_type=jax.ShapeDtypeStruct((_S * 8 * _BT * _TILE,), jnp.float32),
        mesh=mesh,
        scratch_types=[
            pltpu.VMEM((_RQ,), jnp.int32),
            pltpu.VMEM((_RQ,), jnp.int32),
            pltpu.VMEM((_RQ,), jnp.int32),
            pltpu.VMEM((_RQ,), jnp.int32),
            pltpu.VMEM((_RQ,), jnp.int32),
            pltpu.VMEM((_RQ,), jnp.int32),
            pltpu.VMEM((_RQ, 2 * _DIM), jnp.float32),
            pltpu.VMEM((_RQ, 2 * _DIM), jnp.float32),
            pltpu.VMEM((8 * _TILE,), jnp.float32),
            pltpu.VMEM((8 * _TILE,), jnp.float32),
            pltpu.VMEM((8 * _TILE,), jnp.float32),
            pltpu.VMEM((8 * _TILE,), jnp.float32),
            pltpu.SemaphoreType.DMA,
            pltpu.SemaphoreType.DMA,
            pltpu.SemaphoreType.DMA,
            pltpu.SemaphoreType.DMA,
            pltpu.SemaphoreType.DMA,
            pltpu.SemaphoreType.DMA,
            pltpu.SemaphoreType.DMA,
            pltpu.SemaphoreType.DMA,
        ],
        compiler_params=pltpu.CompilerParams(
            use_tc_tiling_on_sc=True, needs_layout_passes=False
        ),
    )


def kernel(x, table):
    xf = (x.astype(jnp.int32).T
          .reshape(_ST, 8, _BT, 128).transpose(0, 2, 1, 3).reshape(-1))
    t2 = table.reshape(_DIM * 1000000 // (2 * _DIM), 2 * _DIM)
    outf = _make_kernel()(xf, t2)
    return (outf.reshape(_S, 8, _BT, 8, 128)
            .transpose(2, 4, 0, 1, 3).reshape(_B, _S, _DIM))


# speedup vs baseline: 1.1584x; 1.1584x over previous
import math

import jax
import jax.numpy as jnp
from jax import lax
from jax.experimental import pallas as pl
from jax.experimental.pallas import tpu as pltpu
from jax.experimental.pallas import tpu_sc as plsc

_DIM = 64
_SCALE = math.sqrt(_DIM)
_NC = 2
_NS = 16
_NW = _NC * _NS
_CH = 512
_LANES = 16
_UNROLL = 8


def _make_kernel(n_rows: int):
    rows_per_w = n_rows // _NW
    n_chunks = rows_per_w // _CH
    mesh = plsc.VectorSubcoreMesh(core_axis_name="c", subcore_axis_name="s")

    def body(x_hbm, table_hbm, out_hbm, idx_v, rows0, rows1, sem0, sem1):
        wid = lax.axis_index("s") * _NC + lax.axis_index("c")
        base = wid * rows_per_w
        pltpu.sync_copy(x_hbm.at[pl.ds(base, rows_per_w)], idx_v)

        bufs = (rows0, rows1)
        sems = (sem0, sem1)

        def gather(c, b):
            return pltpu.make_async_copy(
                table_hbm.at[idx_v.at[pl.ds(c * _CH, _CH)]], bufs[b], sems[b]
            )

        gather(0, 0).start()
        gather(1, 1).start()

        @pl.loop(0, n_chunks, step=2)
        def _ring(g):
            for b in range(2):
                c = g + b
                gather(c, b).wait()

                @pl.loop(0, _CH, step=_UNROLL)
                def _scale(r0):
                    for rr in range(_UNROLL):
                        for j in range(_DIM // _LANES):
                            sl = pl.ds(j * _LANES, _LANES)
                            bufs[b][r0 + rr, sl] = bufs[b][r0 + rr, sl] * _SCALE

                pltpu.sync_copy(bufs[b], out_hbm.at[pl.ds(base + c * _CH, _CH)])

                @pl.when(c < n_chunks - 2)
                def _():
                    gather(c + 2, b).start()

    return pl.kernel(
        body,
        out_type=jax.ShapeDtypeStruct((n_rows, _DIM), jnp.float32),
        mesh=mesh,
        scratch_types=[
            pltpu.VMEM((rows_per_w,), jnp.int32),
            pltpu.VMEM((_CH, _DIM), jnp.float32),
            pltpu.VMEM((_CH, _DIM), jnp.float32),
            pltpu.SemaphoreType.DMA,
            pltpu.SemaphoreType.DMA,
        ],
        compiler_params=pltpu.CompilerParams(use_tc_tiling_on_sc=False),
    )


def kernel(x, table):
    b, s = x.shape
    idx = x.reshape(-1).astype(jnp.int32)
    out = _make_kernel(idx.shape[0])(idx, table)
    return out.reshape(b, s, _DIM)
